# baseline (speedup 1.0000x reference)
import functools

import jax
import jax.numpy as jnp
from jax import lax
from jax.experimental import pallas as pl
from jax.experimental.pallas import tpu as pltpu
from jax.experimental.pallas import tpu_sc as plsc

_EMBED = 64
_SCALE = 8.0
_NC, _NS = 2, 16
_NW = _NC * _NS
_BB = 128

_IOTA = None


@functools.lru_cache(maxsize=None)
def _make_transpose_kernel(embed, vocab):
    n_full = vocab // _BB
    per_w = n_full // _NW
    n_extra = n_full - per_w * _NW

    mesh = plsc.VectorSubcoreMesh(core_axis_name="c", subcore_axis_name="s")

    @functools.partial(
        pl.kernel,
        out_type=jax.ShapeDtypeStruct((vocab // 2 + 32, 2 * embed), jnp.float32),
        mesh=mesh,
        scratch_types=[
            pltpu.VMEM((2, embed, _BB), jnp.float32),
            pltpu.VMEM((2, embed, _BB), jnp.float32),
            pltpu.SemaphoreType.DMA,
            pltpu.SemaphoreType.DMA,
            pltpu.SemaphoreType.DMA,
            pltpu.SemaphoreType.DMA,
        ],
        compiler_params=pltpu.CompilerParams(needs_layout_passes=False),
    )
    def k(tab_t, tail_pad, out_hbm, in_v, out_v, sg0, sg1, sw0, sw1):
        cid = lax.axis_index("c")
        sid = lax.axis_index("s")
        wid = sid * _NC + cid
        my_n = per_w + jnp.where(wid < n_extra, 1, 0)
        iota = lax.iota(jnp.int32, 16)
        sg = (sg0, sg1)
        sw = (sw0, sw1)

        def col_of(u):
            return u * _NW + wid

        def issue_in(u, buf):
            pltpu.async_copy(
                tab_t.at[pl.ds(0, embed), pl.ds(col_of(u) * _BB, _BB)],
                in_v.at[buf],
                sg[buf],
            )

        def wait_in(buf):
            pltpu.make_async_copy(
                tab_t.at[pl.ds(0, embed), pl.ds(0, _BB)], in_v.at[buf], sg[buf]
            ).wait()

        def issue_wb(u, buf):
            pltpu.async_copy(
                out_v.at[buf],
                out_hbm.at[pl.ds(col_of(u) * (_BB // 2), _BB // 2)],
                sw[buf],
            )

        def wait_wb(buf):
            pltpu.make_async_copy(
                out_v.at[buf], out_hbm.at[pl.ds(0, _BB // 2)], sw[buf]
            ).wait()

        def compute(buf):
            @pl.loop(0, _BB // 2, unroll=4)
            def _(r):
                for h in range(2):
                    cv = jnp.zeros((16,), jnp.int32) + (2 * r + h)
                    for q in range(embed // 16):
                        vals = plsc.load_gather(
                            in_v.at[buf], [iota + q * 16, cv]
                        )
                        out_v[buf, r, pl.ds(h * embed + q * 16, 16)] = vals

            return None

        @pl.when(my_n > 0)
        def _():
            issue_in(0, 0)

        @pl.loop(0, per_w + 1, step=2)
        def _(ud):
            for db in range(2):
                u = ud + db

                @pl.when(u < my_n)
                def _():
                    @pl.when(u + 1 < my_n)
                    def _():
                        issue_in(u + 1, 1 - db)

                    wait_in(db)

                    @pl.when(ud >= 2)
                    def _():
                        wait_wb(db)

                    compute(db)
                    issue_wb(u, db)

        wait_wb(0)
        wait_wb(1)

        @pl.when(wid == _NW - 1)
        def _():
            pltpu.sync_copy(tail_pad, in_v.at[0])

            @pl.loop(0, _BB // 2, unroll=4)
            def _(r):
                for h in range(2):
                    cv = jnp.zeros((16,), jnp.int32) + (2 * r + h)
                    for q in range(embed // 16):
                        vals = plsc.load_gather(
                            in_v.at[0], [iota + q * 16, cv]
                        )
                        out_v[0, r, pl.ds(h * embed + q * 16, 16)] = vals

            pltpu.sync_copy(
                out_v.at[0],
                out_hbm.at[pl.ds(n_full * (_BB // 2), _BB // 2)],
            )

    return k


@functools.lru_cache(maxsize=None)
def _make_gather_kernel(n_seq, n_batch, vocab):
    n_bc = n_batch // _BB
    n_blocks = n_seq * n_bc
    blocks_per_w = n_blocks // _NW
    n_super = blocks_per_w // 8

    mesh = plsc.VectorSubcoreMesh(core_axis_name="c", subcore_axis_name="s")

    @functools.partial(
        pl.kernel,
        out_type=jax.ShapeDtypeStruct((n_seq, _EMBED, n_batch), jnp.float32),
        mesh=mesh,
        scratch_types=[
            pltpu.VMEM((n_super, 8, _BB), jnp.int32),
            pltpu.VMEM((2, _BB), jnp.int32),
            pltpu.VMEM((2, _BB, 128), jnp.float32),
            pltpu.VMEM((2, _EMBED, _BB), jnp.float32),
            pltpu.SemaphoreType.DMA,
            pltpu.SemaphoreType.DMA,
            pltpu.SemaphoreType.DMA,
            pltpu.SemaphoreType.DMA,
            pltpu.SemaphoreType.DMA,
        ],
        compiler_params=pltpu.CompilerParams(needs_layout_passes=False),
    )
    def k(idx_hbm, tab_hbm, out_hbm, idx_v, idx2_v, in_v, out_v,
          si, sg0, sg1, sw0, sw1):
        cid = lax.axis_index("c")
        sid = lax.axis_index("s")
        wid = sid * _NC + cid
        sb0 = wid * n_super

        for u in range(n_super):
            sb = sb0 + u
            st = sb // n_bc
            bc = sb % n_bc
            pltpu.async_copy(
                idx_hbm.at[pl.ds(st * 8, 8), pl.ds(bc * _BB, _BB)],
                idx_v.at[u],
                si,
            )
        for u in range(n_super):
            pltpu.make_async_copy(
                idx_hbm.at[pl.ds(0, 8), pl.ds(0, _BB)], idx_v.at[u], si
            ).wait()

        sg = (sg0, sg1)
        sw = (sw0, sw1)
        iota = lax.iota(jnp.int32, 16)
        rowv = [iota + bg * 16 for bg in range(8)]

        def block_coords(j):
            sb = sb0 + (j // 8)
            s = (sb // n_bc) * 8 + (j % 8)
            bc = sb % n_bc
            return s, bc

        def issue_gather(j, buf):
            for bg in range(8):
                sl = pl.ds(bg * 16, 16)
                idx2_v[buf, sl] = lax.shift_right_logical(
                    idx_v[j // 8, j % 8, sl], 1
                )
            pltpu.async_copy(
                tab_hbm.at[idx2_v.at[buf]], in_v.at[buf], sg[buf]
            )

        def wait_gather(buf):
            pltpu.make_async_copy(
                tab_hbm.at[idx2_v.at[buf]], in_v.at[buf], sg[buf]
            ).wait()

        def issue_wb(j, buf):
            s, bc = block_coords(j)
            pltpu.async_copy(
                out_v.at[buf], out_hbm.at[s, :, pl.ds(bc * _BB, _BB)], sw[buf]
            )

        def wait_wb(buf):
            pltpu.make_async_copy(
                out_v.at[buf], out_hbm.at[0, :, pl.ds(0, _BB)], sw[buf]
            ).wait()

        def compute(j, buf):
            halfv = []
            for bg in range(8):
                idxv = idx_v[j // 8, j % 8, pl.ds(bg * 16, 16)]
                halfv.append(lax.shift_left(idxv & 1, 6))

            @pl.loop(0, _EMBED, unroll=4)
            def _(e):
                for bg in range(8):
                    vals = plsc.load_gather(
                        in_v.at[buf], [rowv[bg], halfv[bg] + e]
                    )
                    out_v[buf, e, pl.ds(bg * 16, 16)] = vals * _SCALE

            return None

        issue_gather(0, 0)

        @pl.loop(0, blocks_per_w, step=2)
        def _(jd):
            for db in range(2):
                j = jd + db

                @pl.when(j + 1 < blocks_per_w)
                def _():
                    issue_gather(j + 1, 1 - db)

                wait_gather(db)

                @pl.when(jd >= 2)
                def _():
                    wait_wb(db)

                compute(j, db)
                issue_wb(j, db)

        wait_wb(0)
        wait_wb(1)

    return k


@jax.jit
def kernel(input, table):
    b, s = input.shape
    vocab, embed = table.shape
    idx_t = input.T
    tab_t = table.T
    n_full = vocab // _BB
    tail = lax.slice(tab_t, (0, n_full * _BB), (embed, vocab))
    tail_pad = jnp.pad(tail, ((0, 0), (0, _BB - (vocab - n_full * _BB))))
    tab2_ext = _make_transpose_kernel(embed, vocab)(tab_t, tail_pad)
    out_phys = _make_gather_kernel(s, b, vocab)(idx_t, tab2_ext)
    return out_phys.transpose(2, 0, 1)


# speedup vs baseline: 1.1027x; 1.1027x over previous
import functools

import jax
import jax.numpy as jnp
from jax import lax
from jax.experimental import pallas as pl
from jax.experimental.pallas import tpu as pltpu
from jax.experimental.pallas import tpu_sc as plsc

_EMBED = 64
_SCALE = 8.0
_NC, _NS = 2, 16
_NW = _NC * _NS
_BB = 128

_IOTA = None


@functools.lru_cache(maxsize=None)
def _make_transpose_kernel(embed, vocab):
    n_full = vocab // _BB
    per_w = n_full // _NW
    n_extra = n_full - per_w * _NW

    mesh = plsc.VectorSubcoreMesh(core_axis_name="c", subcore_axis_name="s")

    @functools.partial(
        pl.kernel,
        out_type=jax.ShapeDtypeStruct((vocab // 2 + 32, 2 * embed), jnp.float32),
        mesh=mesh,
        scratch_types=[
            pltpu.VMEM((2, embed, _BB), jnp.float32),
            pltpu.VMEM((2, _BB // 2, 129), jnp.float32),
            pltpu.SemaphoreType.DMA,
            pltpu.SemaphoreType.DMA,
            pltpu.SemaphoreType.DMA,
            pltpu.SemaphoreType.DMA,
        ],
        compiler_params=pltpu.CompilerParams(needs_layout_passes=False),
    )
    def k(tab_t, tail_pad, out_hbm, in_v, out_v, sg0, sg1, sw0, sw1):
        cid = lax.axis_index("c")
        sid = lax.axis_index("s")
        wid = sid * _NC + cid
        my_n = per_w + jnp.where(wid < n_extra, 1, 0)
        iota = lax.iota(jnp.int32, 16)
        sg = (sg0, sg1)
        sw = (sw0, sw1)

        def col_of(u):
            return u * _NW + wid

        def issue_in(u, buf):
            pltpu.async_copy(
                tab_t.at[pl.ds(0, embed), pl.ds(col_of(u) * _BB, _BB)],
                in_v.at[buf],
                sg[buf],
            )

        def wait_in(buf):
            pltpu.make_async_copy(
                tab_t.at[pl.ds(0, embed), pl.ds(0, _BB)], in_v.at[buf], sg[buf]
            ).wait()

        def issue_wb(u, buf):
            pltpu.async_copy(
                out_v.at[buf, pl.ds(0, _BB // 2), pl.ds(0, 2 * embed)],
                out_hbm.at[pl.ds(col_of(u) * (_BB // 2), _BB // 2)],
                sw[buf],
            )

        def wait_wb(buf):
            pltpu.make_async_copy(
                out_v.at[buf, pl.ds(0, _BB // 2), pl.ds(0, 2 * embed)],
                out_hbm.at[pl.ds(0, _BB // 2)],
                sw[buf],
            ).wait()

        rowv = []
        colb = []
        for cg in range(_BB // 16):
            cv = iota + cg * 16
            rowv.append(lax.shift_right_logical(cv, 1))
            colb.append(lax.shift_left(cv & 1, 6))

        def compute(buf):
            @pl.loop(0, embed, unroll=2)
            def _(e):
                for cg in range(_BB // 16):
                    vals = in_v[buf, e, pl.ds(cg * 16, 16)]
                    plsc.store_scatter(
                        out_v.at[buf], [rowv[cg], colb[cg] + e], vals
                    )

            return None

        @pl.when(my_n > 0)
        def _():
            issue_in(0, 0)

        @pl.loop(0, per_w + 1, step=2)
        def _(ud):
            for db in range(2):
                u = ud + db

                @pl.when(u < my_n)
                def _():
                    @pl.when(u + 1 < my_n)
                    def _():
                        issue_in(u + 1, 1 - db)

                    wait_in(db)

                    @pl.when(ud >= 2)
                    def _():
                        wait_wb(db)

                    compute(db)
                    issue_wb(u, db)

        wait_wb(0)
        wait_wb(1)

        @pl.when(wid == _NW - 1)
        def _():
            pltpu.sync_copy(tail_pad, in_v.at[0])
            compute(0)
            pltpu.sync_copy(
                out_v.at[0, pl.ds(0, _BB // 2), pl.ds(0, 2 * embed)],
                out_hbm.at[pl.ds(n_full * (_BB // 2), _BB // 2)],
            )

    return k


@functools.lru_cache(maxsize=None)
def _make_gather_kernel(n_seq, n_batch, vocab):
    n_bc = n_batch // _BB
    n_blocks = n_seq * n_bc
    blocks_per_w = n_blocks // _NW
    n_super = blocks_per_w // 8

    mesh = plsc.VectorSubcoreMesh(core_axis_name="c", subcore_axis_name="s")

    @functools.partial(
        pl.kernel,
        out_type=jax.ShapeDtypeStruct((n_seq, _EMBED, n_batch), jnp.float32),
        mesh=mesh,
        scratch_types=[
            pltpu.VMEM((n_super, 8, _BB), jnp.int32),
            pltpu.VMEM((2, _BB), jnp.int32),
            pltpu.VMEM((2, _BB, 129), jnp.float32),
            pltpu.VMEM((2, _EMBED, _BB), jnp.float32),
            pltpu.SemaphoreType.DMA,
            pltpu.SemaphoreType.DMA,
            pltpu.SemaphoreType.DMA,
            pltpu.SemaphoreType.DMA,
            pltpu.SemaphoreType.DMA,
        ],
        compiler_params=pltpu.CompilerParams(needs_layout_passes=False),
    )
    def k(idx_hbm, tab_hbm, out_hbm, idx_v, idx2_v, in_v, out_v,
          si, sg0, sg1, sw0, sw1):
        cid = lax.axis_index("c")
        sid = lax.axis_index("s")
        wid = sid * _NC + cid
        sb0 = wid * n_super

        for u in range(n_super):
            sb = sb0 + u
            st = sb // n_bc
            bc = sb % n_bc
            pltpu.async_copy(
                idx_hbm.at[pl.ds(st * 8, 8), pl.ds(bc * _BB, _BB)],
                idx_v.at[u],
                si,
            )
        for u in range(n_super):
            pltpu.make_async_copy(
                idx_hbm.at[pl.ds(0, 8), pl.ds(0, _BB)], idx_v.at[u], si
            ).wait()

        sg = (sg0, sg1)
        sw = (sw0, sw1)
        iota = lax.iota(jnp.int32, 16)
        rowv = [iota + bg * 16 for bg in range(8)]

        def block_coords(j):
            sb = sb0 + (j // 8)
            s = (sb // n_bc) * 8 + (j % 8)
            bc = sb % n_bc
            return s, bc

        def issue_gather(j, buf):
            for bg in range(8):
                sl = pl.ds(bg * 16, 16)
                idx2_v[buf, sl] = lax.shift_right_logical(
                    idx_v[j // 8, j % 8, sl], 1
                )
            pltpu.async_copy(
                tab_hbm.at[idx2_v.at[buf]],
                in_v.at[buf, pl.ds(0, _BB), pl.ds(0, 128)],
                sg[buf],
            )

        def wait_gather(buf):
            pltpu.make_async_copy(
                tab_hbm.at[idx2_v.at[buf]],
                in_v.at[buf, pl.ds(0, _BB), pl.ds(0, 128)],
                sg[buf],
            ).wait()

        def issue_wb(j, buf):
            s, bc = block_coords(j)
            pltpu.async_copy(
                out_v.at[buf],
                out_hbm.at[s, :, pl.ds(bc * _BB, _BB)],
                sw[buf],
            )

        def wait_wb(buf):
            pltpu.make_async_copy(
                out_v.at[buf],
                out_hbm.at[0, :, pl.ds(0, _BB)],
                sw[buf],
            ).wait()

        def compute(j, buf):
            halfv = []
            for bg in range(8):
                idxv = idx_v[j // 8, j % 8, pl.ds(bg * 16, 16)]
                halfv.append(lax.shift_left(idxv & 1, 6))

            @pl.loop(0, _EMBED, unroll=2)
            def _(e):
                for bg in range(8):
                    vals = plsc.load_gather(
                        in_v.at[buf], [rowv[bg], halfv[bg] + e]
                    )
                    out_v[buf, e, pl.ds(bg * 16, 16)] = vals * _SCALE

            return None

        issue_gather(0, 0)

        @pl.loop(0, blocks_per_w, step=2)
        def _(jd):
            for db in range(2):
                j = jd + db

                @pl.when(j + 1 < blocks_per_w)
                def _():
                    issue_gather(j + 1, 1 - db)

                wait_gather(db)

                @pl.when(jd >= 2)
                def _():
                    wait_wb(db)

                compute(j, db)
                issue_wb(j, db)

        wait_wb(0)
        wait_wb(1)

    return k


@jax.jit
def kernel(input, table):
    b, s = input.shape
    vocab, embed = table.shape
    idx_t = input.T
    tab_t = table.T
    n_full = vocab // _BB
    tail = lax.slice(tab_t, (0, n_full * _BB), (embed, vocab))
    tail_pad = jnp.pad(tail, ((0, 0), (0, _BB - (vocab - n_full * _BB))))
    tab2_ext = _make_transpose_kernel(embed, vocab)(tab_t, tail_pad)
    out_phys = _make_gather_kernel(s, b, vocab)(idx_t, tab2_ext)
    return out_phys.transpose(2, 0, 1)


# speedup vs baseline: 1.8439x; 1.6721x over previous
import functools

import jax
import jax.numpy as jnp
from jax import lax
from jax.experimental import pallas as pl
from jax.experimental.pallas import tpu as pltpu
from jax.experimental.pallas import tpu_sc as plsc

_EMBED = 64
_SCALE = 8.0
_NC, _NS = 2, 16
_NW = _NC * _NS
_BB = 128

_IOTA = None


@functools.lru_cache(maxsize=None)
def _make_transpose_kernel(embed, vocab):
    n_full = vocab // _BB
    per_w = n_full // _NW
    n_extra = n_full - per_w * _NW

    mesh = plsc.VectorSubcoreMesh(core_axis_name="c", subcore_axis_name="s")

    @functools.partial(
        pl.kernel,
        out_type=jax.ShapeDtypeStruct((vocab // 2 + 32, 2 * embed), jnp.float32),
        mesh=mesh,
        scratch_types=[
            pltpu.VMEM((2, embed, _BB), jnp.float32),
            pltpu.VMEM((2, _BB // 2, 129), jnp.float32),
            pltpu.SemaphoreType.DMA,
            pltpu.SemaphoreType.DMA,
            pltpu.SemaphoreType.DMA,
            pltpu.SemaphoreType.DMA,
        ],
        compiler_params=pltpu.CompilerParams(needs_layout_passes=False),
    )
    def k(tab_t, tail_pad, out_hbm, in_v, out_v, sg0, sg1, sw0, sw1):
        cid = lax.axis_index("c")
        sid = lax.axis_index("s")
        wid = sid * _NC + cid
        my_n = per_w + jnp.where(wid < n_extra, 1, 0)
        iota = lax.iota(jnp.int32, 16)
        sg = (sg0, sg1)
        sw = (sw0, sw1)

        def col_of(u):
            return u * _NW + wid

        def issue_in(u, buf):
            pltpu.async_copy(
                tab_t.at[pl.ds(0, embed), pl.ds(col_of(u) * _BB, _BB)],
                in_v.at[buf],
                sg[buf],
            )

        def wait_in(buf):
            pltpu.make_async_copy(
                tab_t.at[pl.ds(0, embed), pl.ds(0, _BB)], in_v.at[buf], sg[buf]
            ).wait()

        def issue_wb(u, buf):
            pltpu.async_copy(
                out_v.at[buf, pl.ds(0, _BB // 2), pl.ds(0, 2 * embed)],
                out_hbm.at[pl.ds(col_of(u) * (_BB // 2), _BB // 2)],
                sw[buf],
            )

        def wait_wb(buf):
            pltpu.make_async_copy(
                out_v.at[buf, pl.ds(0, _BB // 2), pl.ds(0, 2 * embed)],
                out_hbm.at[pl.ds(0, _BB // 2)],
                sw[buf],
            ).wait()

        rowv = []
        colb = []
        for cg in range(_BB // 16):
            cv = iota + cg * 16
            rowv.append(lax.shift_right_logical(cv, 1))
            colb.append(lax.shift_left(cv & 1, 6))

        def compute(buf):
            @plsc.parallel_loop(0, embed, unroll=2)
            def _(e):
                for cg in range(_BB // 16):
                    vals = in_v[buf, e, pl.ds(cg * 16, 16)]
                    plsc.store_scatter(
                        out_v.at[buf], [rowv[cg], colb[cg] + e], vals
                    )

            return None

        @pl.when(my_n > 0)
        def _():
            issue_in(0, 0)

        @pl.loop(0, per_w + 1, step=2)
        def _(ud):
            for db in range(2):
                u = ud + db

                @pl.when(u < my_n)
                def _():
                    @pl.when(u + 1 < my_n)
                    def _():
                        issue_in(u + 1, 1 - db)

                    wait_in(db)

                    @pl.when(ud >= 2)
                    def _():
                        wait_wb(db)

                    compute(db)
                    issue_wb(u, db)

        wait_wb(0)
        wait_wb(1)

        @pl.when(wid == _NW - 1)
        def _():
            pltpu.sync_copy(tail_pad, in_v.at[0])
            compute(0)
            pltpu.sync_copy(
                out_v.at[0, pl.ds(0, _BB // 2), pl.ds(0, 2 * embed)],
                out_hbm.at[pl.ds(n_full * (_BB // 2), _BB // 2)],
            )

    return k


@functools.lru_cache(maxsize=None)
def _make_gather_kernel(n_seq, n_batch, vocab):
    n_bc = n_batch // _BB
    n_blocks = n_seq * n_bc
    blocks_per_w = n_blocks // _NW
    n_super = blocks_per_w // 8

    mesh = plsc.VectorSubcoreMesh(core_axis_name="c", subcore_axis_name="s")

    @functools.partial(
        pl.kernel,
        out_type=jax.ShapeDtypeStruct((n_seq, _EMBED, n_batch), jnp.float32),
        mesh=mesh,
        scratch_types=[
            pltpu.VMEM((n_super, 8, _BB), jnp.int32),
            pltpu.VMEM((2, _BB), jnp.int32),
            pltpu.VMEM((2, _BB, 129), jnp.float32),
            pltpu.VMEM((2, _EMBED, _BB), jnp.float32),
            pltpu.SemaphoreType.DMA,
            pltpu.SemaphoreType.DMA,
            pltpu.SemaphoreType.DMA,
            pltpu.SemaphoreType.DMA,
            pltpu.SemaphoreType.DMA,
        ],
        compiler_params=pltpu.CompilerParams(needs_layout_passes=False),
    )
    def k(idx_hbm, tab_hbm, out_hbm, idx_v, idx2_v, in_v, out_v,
          si, sg0, sg1, sw0, sw1):
        cid = lax.axis_index("c")
        sid = lax.axis_index("s")
        wid = sid * _NC + cid
        sb0 = wid * n_super

        for u in range(n_super):
            sb = sb0 + u
            st = sb // n_bc
            bc = sb % n_bc
            pltpu.async_copy(
                idx_hbm.at[pl.ds(st * 8, 8), pl.ds(bc * _BB, _BB)],
                idx_v.at[u],
                si,
            )
        for u in range(n_super):
            pltpu.make_async_copy(
                idx_hbm.at[pl.ds(0, 8), pl.ds(0, _BB)], idx_v.at[u], si
            ).wait()

        sg = (sg0, sg1)
        sw = (sw0, sw1)
        iota = lax.iota(jnp.int32, 16)
        rowv = [iota + bg * 16 for bg in range(8)]

        def block_coords(j):
            sb = sb0 + (j // 8)
            s = (sb // n_bc) * 8 + (j % 8)
            bc = sb % n_bc
            return s, bc

        def issue_gather(j, buf):
            for bg in range(8):
                sl = pl.ds(bg * 16, 16)
                idx2_v[buf, sl] = lax.shift_right_logical(
                    idx_v[j // 8, j % 8, sl], 1
                )
            pltpu.async_copy(
                tab_hbm.at[idx2_v.at[buf]],
                in_v.at[buf, pl.ds(0, _BB), pl.ds(0, 128)],
                sg[buf],
            )

        def wait_gather(buf):
            pltpu.make_async_copy(
                tab_hbm.at[idx2_v.at[buf]],
                in_v.at[buf, pl.ds(0, _BB), pl.ds(0, 128)],
                sg[buf],
            ).wait()

        def issue_wb(j, buf):
            s, bc = block_coords(j)
            pltpu.async_copy(
                out_v.at[buf],
                out_hbm.at[s, :, pl.ds(bc * _BB, _BB)],
                sw[buf],
            )

        def wait_wb(buf):
            pltpu.make_async_copy(
                out_v.at[buf],
                out_hbm.at[0, :, pl.ds(0, _BB)],
                sw[buf],
            ).wait()

        def compute(j, buf):
            halfv = []
            for bg in range(8):
                idxv = idx_v[j // 8, j % 8, pl.ds(bg * 16, 16)]
                halfv.append(lax.shift_left(idxv & 1, 6))

            @plsc.parallel_loop(0, _EMBED, unroll=2)
            def _(e):
                for bg in range(8):
                    vals = plsc.load_gather(
                        in_v.at[buf], [rowv[bg], halfv[bg] + e]
                    )
                    out_v[buf, e, pl.ds(bg * 16, 16)] = vals * _SCALE

            return None

        issue_gather(0, 0)

        @pl.loop(0, blocks_per_w, step=2)
        def _(jd):
            for db in range(2):
                j = jd + db

                @pl.when(j + 1 < blocks_per_w)
                def _():
                    issue_gather(j + 1, 1 - db)

                wait_gather(db)

                @pl.when(jd >= 2)
                def _():
                    wait_wb(db)

                compute(j, db)
                issue_wb(j, db)

        wait_wb(0)
        wait_wb(1)

    return k


@jax.jit
def kernel(input, table):
    b, s = input.shape
    vocab, embed = table.shape
    idx_t = input.T
    tab_t = table.T
    n_full = vocab // _BB
    tail = lax.slice(tab_t, (0, n_full * _BB), (embed, vocab))
    tail_pad = jnp.pad(tail, ((0, 0), (0, _BB - (vocab - n_full * _BB))))
    tab2_ext = _make_transpose_kernel(embed, vocab)(tab_t, tail_pad)
    out_phys = _make_gather_kernel(s, b, vocab)(idx_t, tab2_ext)
    return out_phys.transpose(2, 0, 1)


# speedup vs baseline: 1.8444x; 1.0003x over previous
import functools

import jax
import jax.numpy as jnp
from jax import lax
from jax.experimental import pallas as pl
from jax.experimental.pallas import tpu as pltpu
from jax.experimental.pallas import tpu_sc as plsc

_EMBED = 64
_SCALE = 8.0
_NC, _NS = 2, 16
_NW = _NC * _NS
_BB = 128


@functools.lru_cache(maxsize=None)
def _make_transpose_kernel(embed, vocab):
    n_full = vocab // _BB
    per_w = n_full // _NW
    n_extra = n_full - per_w * _NW
    v_pad = (n_full + 1) * _BB

    mesh = plsc.VectorSubcoreMesh(core_axis_name="c", subcore_axis_name="s")

    @functools.partial(
        pl.kernel,
        out_type=jax.ShapeDtypeStruct((v_pad, _BB), jnp.float32),
        mesh=mesh,
        scratch_types=[
            pltpu.VMEM((2, embed, _BB), jnp.float32),
            pltpu.VMEM((2, _BB, 129), jnp.float32),
            pltpu.SemaphoreType.DMA,
            pltpu.SemaphoreType.DMA,
            pltpu.SemaphoreType.DMA,
            pltpu.SemaphoreType.DMA,
        ],
        compiler_params=pltpu.CompilerParams(needs_layout_passes=False),
    )
    def k(tab_t, tail_pad, out_hbm, in_v, out_v, sg0, sg1, sw0, sw1):
        cid = lax.axis_index("c")
        sid = lax.axis_index("s")
        wid = sid * _NC + cid
        my_n = per_w + jnp.where(wid < n_extra, 1, 0)
        iota = lax.iota(jnp.int32, 16)
        sg = (sg0, sg1)
        sw = (sw0, sw1)

        def col_of(u):
            return u * _NW + wid

        def issue_in(u, buf):
            pltpu.async_copy(
                tab_t.at[pl.ds(0, embed), pl.ds(col_of(u) * _BB, _BB)],
                in_v.at[buf],
                sg[buf],
            )

        def wait_in(buf):
            pltpu.make_async_copy(
                tab_t.at[pl.ds(0, embed), pl.ds(0, _BB)], in_v.at[buf], sg[buf]
            ).wait()

        def issue_wb(u, buf):
            pltpu.async_copy(
                out_v.at[buf, pl.ds(0, _BB), pl.ds(0, _BB)],
                out_hbm.at[pl.ds(col_of(u) * _BB, _BB)],
                sw[buf],
            )

        def wait_wb(buf):
            pltpu.make_async_copy(
                out_v.at[buf, pl.ds(0, _BB), pl.ds(0, _BB)],
                out_hbm.at[pl.ds(0, _BB)],
                sw[buf],
            ).wait()

        rowv = [iota + cg * 16 for cg in range(_BB // 16)]

        def compute(buf):
            @plsc.parallel_loop(0, embed, unroll=2)
            def _(e):
                ev = jnp.zeros((16,), jnp.int32) + e
                for cg in range(_BB // 16):
                    vals = in_v[buf, e, pl.ds(cg * 16, 16)]
                    plsc.store_scatter(out_v.at[buf], [rowv[cg], ev], vals)

            return None

        @pl.when(my_n > 0)
        def _():
            issue_in(0, 0)

        @pl.loop(0, per_w + 1, step=2)
        def _(ud):
            for db in range(2):
                u = ud + db

                @pl.when(u < my_n)
                def _():
                    @pl.when(u + 1 < my_n)
                    def _():
                        issue_in(u + 1, 1 - db)

                    wait_in(db)

                    @pl.when(ud >= 2)
                    def _():
                        wait_wb(db)

                    compute(db)
                    issue_wb(u, db)

        wait_wb(0)
        wait_wb(1)

        @pl.when(wid == _NW - 1)
        def _():
            pltpu.sync_copy(tail_pad, in_v.at[0])
            compute(0)
            pltpu.sync_copy(
                out_v.at[0, pl.ds(0, _BB), pl.ds(0, _BB)],
                out_hbm.at[pl.ds(n_full * _BB, _BB)],
            )

    return k


@functools.lru_cache(maxsize=None)
def _make_gather_kernel(n_seq, n_batch):
    n_bc = n_batch // _BB
    n_blocks = n_seq * n_bc
    blocks_per_w = n_blocks // _NW
    n_super = blocks_per_w // 8

    mesh = plsc.VectorSubcoreMesh(core_axis_name="c", subcore_axis_name="s")

    @functools.partial(
        pl.kernel,
        out_type=jax.ShapeDtypeStruct((n_seq, _EMBED, n_batch), jnp.float32),
        mesh=mesh,
        scratch_types=[
            pltpu.VMEM((n_super, 8, _BB), jnp.int32),
            pltpu.VMEM((2, _BB, 129), jnp.float32),
            pltpu.VMEM((2, _EMBED, _BB), jnp.float32),
            pltpu.SemaphoreType.DMA,
            pltpu.SemaphoreType.DMA,
            pltpu.SemaphoreType.DMA,
            pltpu.SemaphoreType.DMA,
            pltpu.SemaphoreType.DMA,
        ],
        compiler_params=pltpu.CompilerParams(needs_layout_passes=False),
    )
    def k(idx_hbm, tab_hbm, out_hbm, idx_v, in_v, out_v, si, sg0, sg1, sw0, sw1):
        cid = lax.axis_index("c")
        sid = lax.axis_index("s")
        wid = sid * _NC + cid
        sb0 = wid * n_super

        for u in range(n_super):
            sb = sb0 + u
            st = sb // n_bc
            bc = sb % n_bc
            pltpu.async_copy(
                idx_hbm.at[pl.ds(st * 8, 8), pl.ds(bc * _BB, _BB)],
                idx_v.at[u],
                si,
            )
        for u in range(n_super):
            pltpu.make_async_copy(
                idx_hbm.at[pl.ds(0, 8), pl.ds(0, _BB)], idx_v.at[u], si
            ).wait()

        sg = (sg0, sg1)
        sw = (sw0, sw1)
        iota = lax.iota(jnp.int32, 16)
        rowv = [iota + bg * 16 for bg in range(8)]

        def block_coords(j):
            sb = sb0 + (j // 8)
            s = (sb // n_bc) * 8 + (j % 8)
            bc = sb % n_bc
            return s, bc

        def issue_gather(j, buf):
            pltpu.async_copy(
                tab_hbm.at[idx_v.at[j // 8, j % 8]],
                in_v.at[buf, pl.ds(0, _BB), pl.ds(0, _BB)],
                sg[buf],
            )

        def wait_gather(buf):
            pltpu.make_async_copy(
                tab_hbm.at[idx_v.at[0, 0]],
                in_v.at[buf, pl.ds(0, _BB), pl.ds(0, _BB)],
                sg[buf],
            ).wait()

        def issue_wb(j, buf):
            s, bc = block_coords(j)
            pltpu.async_copy(
                out_v.at[buf],
                out_hbm.at[s, :, pl.ds(bc * _BB, _BB)],
                sw[buf],
            )

        def wait_wb(buf):
            pltpu.make_async_copy(
                out_v.at[buf],
                out_hbm.at[0, :, pl.ds(0, _BB)],
                sw[buf],
            ).wait()

        def compute(buf):
            @plsc.parallel_loop(0, _EMBED, unroll=2)
            def _(e):
                ev = jnp.zeros((16,), jnp.int32) + e
                for bg in range(8):
                    vals = plsc.load_gather(in_v.at[buf], [rowv[bg], ev])
                    out_v[buf, e, pl.ds(bg * 16, 16)] = vals * _SCALE

            return None

        issue_gather(0, 0)

        @pl.loop(0, blocks_per_w, step=2)
        def _(jd):
            for db in range(2):
                j = jd + db

                @pl.when(j + 1 < blocks_per_w)
                def _():
                    issue_gather(j + 1, 1 - db)

                wait_gather(db)

                @pl.when(jd >= 2)
                def _():
                    wait_wb(db)

                compute(db)
                issue_wb(j, db)

        wait_wb(0)
        wait_wb(1)

    return k


@jax.jit
def kernel(input, table):
    b, s = input.shape
    vocab, embed = table.shape
    idx_t = input.T
    tab_t = table.T
    n_full = vocab // _BB
    tail = lax.slice(tab_t, (0, n_full * _BB), (embed, vocab))
    tail_pad = jnp.pad(tail, ((0, 0), (0, _BB - (vocab - n_full * _BB))))
    tab_pad = _make_transpose_kernel(embed, vocab)(tab_t, tail_pad)
    out_phys = _make_gather_kernel(s, b)(idx_t, tab_pad)
    return out_phys.transpose(2, 0, 1)


# speedup vs baseline: 1.8488x; 1.0024x over previous
import functools

import jax
import jax.numpy as jnp
from jax import lax
from jax.experimental import pallas as pl
from jax.experimental.pallas import tpu as pltpu
from jax.experimental.pallas import tpu_sc as plsc

_EMBED = 64
_SCALE = 8.0
_NC, _NS = 2, 16
_NW = _NC * _NS
_BB = 128


@functools.lru_cache(maxsize=None)
def _make_transpose_kernel(embed, vocab):
    n_full = vocab // _BB
    per_w = n_full // _NW
    n_extra = n_full - per_w * _NW
    v_pad = (n_full + 1) * _BB

    mesh = plsc.VectorSubcoreMesh(core_axis_name="c", subcore_axis_name="s")

    @functools.partial(
        pl.kernel,
        out_type=jax.ShapeDtypeStruct((v_pad, _BB), jnp.float32),
        mesh=mesh,
        scratch_types=[
            pltpu.VMEM((2, embed, _BB), jnp.float32),
            pltpu.VMEM((2, _BB, 129), jnp.float32),
            pltpu.SemaphoreType.DMA,
            pltpu.SemaphoreType.DMA,
            pltpu.SemaphoreType.DMA,
            pltpu.SemaphoreType.DMA,
        ],
        compiler_params=pltpu.CompilerParams(needs_layout_passes=False),
    )
    def k(tab_t, tail_pad, out_hbm, in_v, out_v, sg0, sg1, sw0, sw1):
        cid = lax.axis_index("c")
        sid = lax.axis_index("s")
        wid = sid * _NC + cid
        my_n = per_w + jnp.where(wid < n_extra, 1, 0)
        iota = lax.iota(jnp.int32, 16)
        sg = (sg0, sg1)
        sw = (sw0, sw1)

        def col_of(u):
            return u * _NW + wid

        def issue_in(u, buf):
            pltpu.async_copy(
                tab_t.at[pl.ds(0, embed), pl.ds(col_of(u) * _BB, _BB)],
                in_v.at[buf],
                sg[buf],
            )

        def wait_in(buf):
            pltpu.make_async_copy(
                tab_t.at[pl.ds(0, embed), pl.ds(0, _BB)], in_v.at[buf], sg[buf]
            ).wait()

        def issue_wb(u, buf):
            pltpu.async_copy(
                out_v.at[buf, pl.ds(0, _BB), pl.ds(0, _BB)],
                out_hbm.at[pl.ds(col_of(u) * _BB, _BB)],
                sw[buf],
            )

        def wait_wb(buf):
            pltpu.make_async_copy(
                out_v.at[buf, pl.ds(0, _BB), pl.ds(0, _BB)],
                out_hbm.at[pl.ds(0, _BB)],
                sw[buf],
            ).wait()

        rowv = [iota + cg * 16 for cg in range(_BB // 16)]

        def compute(buf):
            @plsc.parallel_loop(0, embed, unroll=4)
            def _(e):
                ev = jnp.zeros((16,), jnp.int32) + e
                for cg in range(_BB // 16):
                    vals = in_v[buf, e, pl.ds(cg * 16, 16)]
                    plsc.store_scatter(out_v.at[buf], [rowv[cg], ev], vals)

            return None

        @pl.when(my_n > 0)
        def _():
            issue_in(0, 0)

        @pl.loop(0, per_w + 1, step=2)
        def _(ud):
            for db in range(2):
                u = ud + db

                @pl.when(u < my_n)
                def _():
                    @pl.when(u + 1 < my_n)
                    def _():
                        issue_in(u + 1, 1 - db)

                    wait_in(db)

                    @pl.when(ud >= 2)
                    def _():
                        wait_wb(db)

                    compute(db)
                    issue_wb(u, db)

        wait_wb(0)
        wait_wb(1)

        @pl.when(wid == _NW - 1)
        def _():
            pltpu.sync_copy(tail_pad, in_v.at[0])
            compute(0)
            pltpu.sync_copy(
                out_v.at[0, pl.ds(0, _BB), pl.ds(0, _BB)],
                out_hbm.at[pl.ds(n_full * _BB, _BB)],
            )

    return k


@functools.lru_cache(maxsize=None)
def _make_gather_kernel(n_seq, n_batch):
    n_bc = n_batch // _BB
    n_blocks = n_seq * n_bc
    blocks_per_w = n_blocks // _NW
    n_super = blocks_per_w // 8

    mesh = plsc.VectorSubcoreMesh(core_axis_name="c", subcore_axis_name="s")

    @functools.partial(
        pl.kernel,
        out_type=jax.ShapeDtypeStruct((n_seq, _EMBED, n_batch), jnp.float32),
        mesh=mesh,
        scratch_types=[
            pltpu.VMEM((n_super, 8, _BB), jnp.int32),
            pltpu.VMEM((2, _BB, 129), jnp.float32),
            pltpu.VMEM((2, _EMBED, _BB), jnp.float32),
            pltpu.SemaphoreType.DMA,
            pltpu.SemaphoreType.DMA,
            pltpu.SemaphoreType.DMA,
            pltpu.SemaphoreType.DMA,
            pltpu.SemaphoreType.DMA,
        ],
        compiler_params=pltpu.CompilerParams(needs_layout_passes=False),
    )
    def k(idx_hbm, tab_hbm, out_hbm, idx_v, in_v, out_v, si, sg0, sg1, sw0, sw1):
        cid = lax.axis_index("c")
        sid = lax.axis_index("s")
        wid = sid * _NC + cid
        sb0 = wid * n_super

        for u in range(n_super):
            sb = sb0 + u
            st = sb // n_bc
            bc = sb % n_bc
            pltpu.async_copy(
                idx_hbm.at[pl.ds(st * 8, 8), pl.ds(bc * _BB, _BB)],
                idx_v.at[u],
                si,
            )
        for u in range(n_super):
            pltpu.make_async_copy(
                idx_hbm.at[pl.ds(0, 8), pl.ds(0, _BB)], idx_v.at[u], si
            ).wait()

        sg = (sg0, sg1)
        sw = (sw0, sw1)
        iota = lax.iota(jnp.int32, 16)
        rowv = [iota + bg * 16 for bg in range(8)]

        def block_coords(j):
            sb = sb0 + (j // 8)
            s = (sb // n_bc) * 8 + (j % 8)
            bc = sb % n_bc
            return s, bc

        def issue_gather(j, buf):
            pltpu.async_copy(
                tab_hbm.at[idx_v.at[j // 8, j % 8]],
                in_v.at[buf, pl.ds(0, _BB), pl.ds(0, _BB)],
                sg[buf],
            )

        def wait_gather(buf):
            pltpu.make_async_copy(
                tab_hbm.at[idx_v.at[0, 0]],
                in_v.at[buf, pl.ds(0, _BB), pl.ds(0, _BB)],
                sg[buf],
            ).wait()

        def issue_wb(j, buf):
            s, bc = block_coords(j)
            pltpu.async_copy(
                out_v.at[buf],
                out_hbm.at[s, :, pl.ds(bc * _BB, _BB)],
                sw[buf],
            )

        def wait_wb(buf):
            pltpu.make_async_copy(
                out_v.at[buf],
                out_hbm.at[0, :, pl.ds(0, _BB)],
                sw[buf],
            ).wait()

        def compute(buf):
            @plsc.parallel_loop(0, _EMBED, unroll=4)
            def _(e):
                ev = jnp.zeros((16,), jnp.int32) + e
                for bg in range(8):
                    vals = plsc.load_gather(in_v.at[buf], [rowv[bg], ev])
                    out_v[buf, e, pl.ds(bg * 16, 16)] = vals * _SCALE

            return None

        issue_gather(0, 0)

        @pl.loop(0, blocks_per_w, step=2)
        def _(jd):
            for db in range(2):
                j = jd + db

                @pl.when(j + 1 < blocks_per_w)
                def _():
                    issue_gather(j + 1, 1 - db)

                wait_gather(db)

                @pl.when(jd >= 2)
                def _():
                    wait_wb(db)

                compute(db)
                issue_wb(j, db)

        wait_wb(0)
        wait_wb(1)

    return k


@jax.jit
def kernel(input, table):
    b, s = input.shape
    vocab, embed = table.shape
    idx_t = input.T
    tab_t = table.T
    n_full = vocab // _BB
    tail = lax.slice(tab_t, (0, n_full * _BB), (embed, vocab))
    tail_pad = jnp.pad(tail, ((0, 0), (0, _BB - (vocab - n_full * _BB))))
    tab_pad = _make_transpose_kernel(embed, vocab)(tab_t, tail_pad)
    out_phys = _make_gather_kernel(s, b)(idx_t, tab_pad)
    return out_phys.transpose(2, 0, 1)


# speedup vs baseline: 2.2401x; 1.2117x over previous
import functools

import jax
import jax.numpy as jnp
from jax import lax
from jax.experimental import pallas as pl
from jax.experimental.pallas import tpu as pltpu
from jax.experimental.pallas import tpu_sc as plsc

_EMBED = 64
_SCALE = 8.0
_NC, _NS = 2, 16
_NW = _NC * _NS
_BB = 128


@functools.lru_cache(maxsize=None)
def _make_transpose_kernel(embed, vocab):
    n_full = vocab // _BB
    per_w = n_full // _NW
    n_extra = n_full - per_w * _NW
    v_pad = (n_full + 1) * _BB

    mesh = plsc.VectorSubcoreMesh(core_axis_name="c", subcore_axis_name="s")

    @functools.partial(
        pl.kernel,
        out_type=jax.ShapeDtypeStruct((v_pad, _BB), jnp.float32),
        mesh=mesh,
        scratch_types=[
            pltpu.VMEM((2, embed, _BB), jnp.float32),
            pltpu.VMEM((2, _BB, 129), jnp.float32),
            pltpu.SemaphoreType.DMA,
            pltpu.SemaphoreType.DMA,
            pltpu.SemaphoreType.DMA,
            pltpu.SemaphoreType.DMA,
        ],
        compiler_params=pltpu.CompilerParams(needs_layout_passes=False),
    )
    def k(tab_t, tail_pad, out_hbm, in_v, out_v, sg0, sg1, sw0, sw1):
        cid = lax.axis_index("c")
        sid = lax.axis_index("s")
        wid = sid * _NC + cid
        my_n = per_w + jnp.where(wid < n_extra, 1, 0)
        iota = lax.iota(jnp.int32, 16)
        sg = (sg0, sg1)
        sw = (sw0, sw1)

        def col_of(u):
            return u * _NW + wid

        def issue_in(u, buf):
            pltpu.async_copy(
                tab_t.at[pl.ds(0, embed), pl.ds(col_of(u) * _BB, _BB)],
                in_v.at[buf],
                sg[buf],
            )

        def wait_in(buf):
            pltpu.make_async_copy(
                tab_t.at[pl.ds(0, embed), pl.ds(0, _BB)], in_v.at[buf], sg[buf]
            ).wait()

        def issue_wb(u, buf):
            pltpu.async_copy(
                out_v.at[buf, pl.ds(0, _BB), pl.ds(0, _BB)],
                out_hbm.at[pl.ds(col_of(u) * _BB, _BB)],
                sw[buf],
            )

        def wait_wb(buf):
            pltpu.make_async_copy(
                out_v.at[buf, pl.ds(0, _BB), pl.ds(0, _BB)],
                out_hbm.at[pl.ds(0, _BB)],
                sw[buf],
            ).wait()

        rowv = [iota + cg * 16 for cg in range(_BB // 16)]

        def compute(buf):
            @plsc.parallel_loop(0, embed, unroll=4)
            def _(e):
                ev = jnp.zeros((16,), jnp.int32) + e
                for cg in range(_BB // 16):
                    vals = in_v[buf, e, pl.ds(cg * 16, 16)]
                    plsc.store_scatter(out_v.at[buf], [rowv[cg], ev], vals)

            return None

        @pl.when(my_n > 0)
        def _():
            issue_in(0, 0)

        @pl.loop(0, per_w + 1, step=2)
        def _(ud):
            for db in range(2):
                u = ud + db

                @pl.when(u < my_n)
                def _():
                    @pl.when(u + 1 < my_n)
                    def _():
                        issue_in(u + 1, 1 - db)

                    wait_in(db)

                    @pl.when(ud >= 2)
                    def _():
                        wait_wb(db)

                    compute(db)
                    issue_wb(u, db)

        wait_wb(0)
        wait_wb(1)

        @pl.when(wid == _NW - 1)
        def _():
            pltpu.sync_copy(tail_pad, in_v.at[0])
            compute(0)
            pltpu.sync_copy(
                out_v.at[0, pl.ds(0, _BB), pl.ds(0, _BB)],
                out_hbm.at[pl.ds(n_full * _BB, _BB)],
            )

    return k


@functools.lru_cache(maxsize=None)
def _make_gather_kernel(n_seq, n_batch):
    n_bc = n_batch // _BB
    n_blocks = n_seq * n_bc
    blocks_per_w = n_blocks // _NW
    n_super = blocks_per_w // 8

    mesh = plsc.VectorSubcoreMesh(core_axis_name="c", subcore_axis_name="s")

    @functools.partial(
        pl.kernel,
        out_type=jax.ShapeDtypeStruct((n_seq, _EMBED, n_batch), jnp.float32),
        mesh=mesh,
        scratch_types=[
            pltpu.VMEM((n_super, 8, _BB), jnp.int32),
            pltpu.VMEM((2, _BB), jnp.int32),
            pltpu.VMEM((2, _BB, 129), jnp.float32),
            pltpu.VMEM((2, _EMBED, _BB), jnp.float32),
            pltpu.SemaphoreType.DMA,
            pltpu.SemaphoreType.DMA,
            pltpu.SemaphoreType.DMA,
            pltpu.SemaphoreType.DMA,
            pltpu.SemaphoreType.DMA,
        ],
        compiler_params=pltpu.CompilerParams(needs_layout_passes=False),
    )
    def k(idx_hbm, tab_hbm, out_hbm, idx_v, idx2_v, in_v, out_v,
          si, sg0, sg1, sw0, sw1):
        cid = lax.axis_index("c")
        sid = lax.axis_index("s")
        wid = sid * _NC + cid
        sb0 = wid * n_super

        for u in range(n_super):
            sb = sb0 + u
            st = sb // n_bc
            bc = sb % n_bc
            pltpu.async_copy(
                idx_hbm.at[pl.ds(st * 8, 8), pl.ds(bc * _BB, _BB)],
                idx_v.at[u],
                si,
            )
        for u in range(n_super):
            pltpu.make_async_copy(
                idx_hbm.at[pl.ds(0, 8), pl.ds(0, _BB)], idx_v.at[u], si
            ).wait()

        sg = (sg0, sg1)
        sw = (sw0, sw1)
        iota = lax.iota(jnp.int32, 16)
        rowv = [iota + bg * 16 for bg in range(8)]

        def block_coords(j):
            sb = sb0 + (j // 8)
            s = (sb // n_bc) * 8 + (j % 8)
            bc = sb % n_bc
            return s, bc

        def issue_gather(j, buf):
            for bg in range(8):
                sl = pl.ds(bg * 16, 16)
                idx2_v[buf, sl] = lax.shift_right_logical(
                    idx_v[j // 8, j % 8, sl], 1
                )
            pltpu.async_copy(
                tab_hbm.at[idx2_v.at[buf]],
                in_v.at[buf, pl.ds(0, _BB), pl.ds(0, _BB)],
                sg[buf],
            )

        def wait_gather(buf):
            pltpu.make_async_copy(
                tab_hbm.at[idx2_v.at[buf]],
                in_v.at[buf, pl.ds(0, _BB), pl.ds(0, _BB)],
                sg[buf],
            ).wait()

        def issue_wb(j, buf):
            s, bc = block_coords(j)
            pltpu.async_copy(
                out_v.at[buf],
                out_hbm.at[s, :, pl.ds(bc * _BB, _BB)],
                sw[buf],
            )

        def wait_wb(buf):
            pltpu.make_async_copy(
                out_v.at[buf],
                out_hbm.at[0, :, pl.ds(0, _BB)],
                sw[buf],
            ).wait()

        def compute(j, buf):
            halfv = []
            for bg in range(8):
                idxv = idx_v[j // 8, j % 8, pl.ds(bg * 16, 16)]
                halfv.append(lax.shift_left(idxv & 1, 6))

            @plsc.parallel_loop(0, _EMBED, unroll=4)
            def _(e):
                for bg in range(8):
                    vals = plsc.load_gather(
                        in_v.at[buf], [rowv[bg], halfv[bg] + e]
                    )
                    out_v[buf, e, pl.ds(bg * 16, 16)] = vals * _SCALE

            return None

        issue_gather(0, 0)

        @pl.loop(0, blocks_per_w, step=2)
        def _(jd):
            for db in range(2):
                j = jd + db

                @pl.when(j + 1 < blocks_per_w)
                def _():
                    issue_gather(j + 1, 1 - db)

                wait_gather(db)

                @pl.when(jd >= 2)
                def _():
                    wait_wb(db)

                compute(j, db)
                issue_wb(j, db)

        wait_wb(0)
        wait_wb(1)

    return k


@jax.jit
def kernel(input, table):
    b, s = input.shape
    vocab, embed = table.shape
    idx_t = input.T
    tab2 = table.reshape(vocab // 2, 2 * embed)
    out_phys = _make_gather_kernel(s, b)(idx_t, tab2)
    return out_phys.transpose(2, 0, 1)
